# baseline (speedup 1.0000x reference)
import jax
import jax.numpy as jnp
from jax import lax
from jax.experimental import pallas as pl
from jax.experimental.pallas import tpu as pltpu
from jax.experimental.pallas import tpu_sc as plsc

BATCH = 16384
D_NUM = 13
N_CAT = 26
D_TOKEN = 32
N_TOK = D_NUM + N_CAT
TABLE_ROWS = 2600000
LINE = 128
TABLE_LINES = TABLE_ROWS * D_TOKEN // LINE

_info = plsc.get_sparse_core_info()
NC, NS, L = _info.num_cores, _info.num_subcores, _info.num_lanes
NW = NC * NS
BPW = BATCH // NW

C = 16
G2 = BPW // (2 * C)
R = C * N_CAT
N_DMA = (R + LINE - 1) // LINE
CHUNK_LINES = C * N_TOK // 4
QPAD = 432


def _encoder_body(xnumf_hbm, xcatf_hbm, weightf_hbm, table_hbm, biasf_hbm,
                  offs_hbm, out_hbm,
                  xcatf_v, xnumf_v, offs_v, weight_v, bias_v, idx_v, qmod_v,
                  rows_v, stage_v, sem):
    wid = lax.axis_index("s") * NC + lax.axis_index("c")

    pltpu.sync_copy(offs_hbm, offs_v)
    pltpu.sync_copy(weightf_hbm, weight_v)
    pltpu.sync_copy(biasf_hbm, bias_v)

    zero16 = jnp.zeros((L,), jnp.int32)
    for t in range((LINE - R % LINE) // L):
        idx_v[N_DMA - 1, pl.ds(R % LINE + t * L, L)] = zero16

    def sub_chunk(base, stage0):
        pltpu.sync_copy(xcatf_hbm.at[pl.ds(base * N_CAT, R)], xcatf_v)
        pltpu.sync_copy(xnumf_hbm.at[pl.ds(base * L, C * L)], xnumf_v)

        for v in range(R // L):
            p = v * L
            full = xcatf_v[pl.ds(p, L)] + offs_v[pl.ds(p, L)]
            idx_v[p // LINE, pl.ds(p % LINE, L)] = lax.shift_right_logical(
                full, 2)
            qmod_v[pl.ds(p, L)] = lax.bitwise_and(full, 3)

        handles = [
            pltpu.async_copy(table_hbm.at[idx_v.at[r]],
                             rows_v.at[pl.ds(r * LINE, LINE)], sem)
            for r in range(N_DMA)
        ]
        for h in handles:
            h.wait()

        wnum = [weight_v[pl.ds(d * D_TOKEN + h * L, L)]
                for d in range(D_NUM) for h in range(2)]
        bnum = [bias_v[pl.ds(d * D_TOKEN + h * L, L)]
                for d in range(D_NUM) for h in range(2)]

        def num_body(c4, carry2):
            for cp in range(4):
                xrow = xnumf_v[pl.ds(c4 * 4 * L + cp * L, L)]
                for d in range(D_NUM):
                    sv = jnp.full((L,), xrow[d], jnp.float32)
                    tok = 39 * cp + d
                    for h in range(2):
                        stage_v[stage0 + 39 * c4 + tok // 4,
                                pl.ds((tok % 4) * D_TOKEN + h * L, L)] = (
                            sv * wnum[2 * d + h] + bnum[2 * d + h])
            return carry2

        lax.fori_loop(0, C // 4, num_body, 0)

        bcat = [bias_v[pl.ds((D_NUM + j) * D_TOKEN + h * L, L)]
                for j in range(N_CAT) for h in range(2)]

        def cat_body(c4, carry2):
            p0 = c4 * 4 * N_CAT
            qv = [qmod_v[pl.ds(p0 + t * L, L)] for t in range(7)]
            for cp in range(4):
                for j in range(N_CAT):
                    i = cp * N_CAT + j
                    bq = jnp.full((L,), qv[i // L][i % L], jnp.int32)
                    flo = lax.bitwise_and(bq, 1).astype(jnp.float32)
                    fhi = lax.shift_right_logical(bq, 1).astype(jnp.float32)
                    tok = 39 * cp + D_NUM + j
                    for h in range(2):
                        v0 = rows_v[p0 + i, pl.ds(h * L, L)]
                        v1 = rows_v[p0 + i, pl.ds(D_TOKEN + h * L, L)]
                        v2 = rows_v[p0 + i, pl.ds(2 * D_TOKEN + h * L, L)]
                        v3 = rows_v[p0 + i, pl.ds(3 * D_TOKEN + h * L, L)]
                        s01 = v0 + flo * (v1 - v0)
                        s23 = v2 + flo * (v3 - v2)
                        v = s01 + fhi * (s23 - s01)
                        stage_v[stage0 + 39 * c4 + tok // 4,
                                pl.ds((tok % 4) * D_TOKEN + h * L, L)] = (
                            v + bcat[2 * j + h])
            return carry2

        lax.fori_loop(0, C // 4, cat_body, 0)

    def chunk_body(g2, carry):
        base = wid * BPW + g2 * 2 * C
        sub_chunk(base, 0)
        sub_chunk(base + C, CHUNK_LINES)
        row0 = pl.multiple_of(base * N_TOK // 4, 8)
        pltpu.sync_copy(stage_v, out_hbm.at[pl.ds(row0, 2 * CHUNK_LINES)])
        return carry

    lax.fori_loop(0, G2, chunk_body, 0)


@jax.jit
def _encoder(x_numf, x_catf, weightf, table2, biasf, offs_tile):
    mesh = plsc.VectorSubcoreMesh(core_axis_name="c", subcore_axis_name="s")
    f = pl.kernel(
        _encoder_body, mesh=mesh,
        out_type=jax.ShapeDtypeStruct((BATCH * N_TOK // 4, LINE), jnp.float32),
        scratch_types=[
            pltpu.VMEM((R,), jnp.int32),
            pltpu.VMEM((C * L,), jnp.float32),
            pltpu.VMEM((R,), jnp.int32),
            pltpu.VMEM((D_NUM * D_TOKEN,), jnp.float32),
            pltpu.VMEM((N_TOK * D_TOKEN,), jnp.float32),
            pltpu.VMEM((N_DMA, LINE), jnp.int32),
            pltpu.VMEM((QPAD,), jnp.int32),
            pltpu.VMEM((N_DMA * LINE, LINE), jnp.float32),
            pltpu.VMEM((2 * CHUNK_LINES, LINE), jnp.float32),
            pltpu.SemaphoreType.DMA,
        ],
    )
    return f(x_numf, x_catf, weightf, table2, biasf, offs_tile)


def kernel(x_num, x_cat, weight, cat_table, tab_bias, category_offsets):
    x_numf = jnp.pad(x_num, ((0, 0), (0, L - D_NUM))).reshape(BATCH * L)
    x_catf = x_cat.reshape(BATCH * N_CAT)
    table2 = cat_table.reshape(TABLE_LINES, LINE)
    weightf = weight.reshape(D_NUM * D_TOKEN)
    biasf = tab_bias.reshape(N_TOK * D_TOKEN)
    offs_tile = jnp.tile(category_offsets.astype(jnp.int32), C)
    out = _encoder(x_numf, x_catf, weightf, table2, biasf, offs_tile)
    return out.reshape(BATCH, N_TOK, D_TOKEN)


# speedup vs baseline: 2.1878x; 2.1878x over previous
import jax
import jax.numpy as jnp
from jax import lax
from jax.experimental import pallas as pl
from jax.experimental.pallas import tpu as pltpu
from jax.experimental.pallas import tpu_sc as plsc

BATCH = 16384
D_NUM = 13
N_CAT = 26
D_TOKEN = 32
N_TOK = D_NUM + N_CAT
TABLE_ROWS = 2600000
LINE = 128
TABLE_LINES = TABLE_ROWS * D_TOKEN // LINE

_info = plsc.get_sparse_core_info()
NC, NS, L = _info.num_cores, _info.num_subcores, _info.num_lanes
NW = NC * NS
BPW = BATCH // NW

C = 16
G2 = BPW // (2 * C)
R = C * N_CAT
N_DMA = (R + LINE - 1) // LINE
CHUNK_LINES = C * N_TOK // 4
QPAD = 432


def _encoder_body(xnumf_hbm, xcatf_hbm, weightf_hbm, table_hbm, biasf_hbm,
                  offs_hbm, out_hbm,
                  xcatf_v, xnumf_v, offs_v, weight_v, bias_v, idx_v, qmod_v,
                  rows_v, stage_v, sem):
    wid = lax.axis_index("s") * NC + lax.axis_index("c")

    pltpu.sync_copy(offs_hbm, offs_v)
    pltpu.sync_copy(weightf_hbm, weight_v)
    pltpu.sync_copy(biasf_hbm, bias_v)


    def sub_chunk(base, stage0):
        pltpu.sync_copy(xcatf_hbm.at[pl.ds(base * N_CAT, R)], xcatf_v)
        pltpu.sync_copy(xnumf_hbm.at[pl.ds(base * L, C * L)], xnumf_v)

        for v in range(R // L):
            p = v * L
            full = xcatf_v[pl.ds(p, L)] + offs_v[pl.ds(p, L)]
            idx_v[p // LINE, pl.ds(p % LINE, L)] = lax.shift_right_logical(
                full, 2)
            qmod_v[pl.ds(p, L)] = lax.bitwise_and(full, 3)

        handles = [
            pltpu.async_copy(table_hbm.at[idx_v.at[r]],
                             rows_v.at[pl.ds(r * LINE, LINE)], sem)
            for r in range(N_DMA - 1)
        ]
        handles.append(
            pltpu.async_copy(
                table_hbm.at[idx_v.at[N_DMA - 1, pl.ds(0, R % LINE)]],
                rows_v.at[pl.ds((N_DMA - 1) * LINE, R % LINE)], sem))
        for h in handles:
            h.wait()

        wnum = [weight_v[pl.ds(d * D_TOKEN + h * L, L)]
                for d in range(D_NUM) for h in range(2)]
        bnum = [bias_v[pl.ds(d * D_TOKEN + h * L, L)]
                for d in range(D_NUM) for h in range(2)]

        def num_body(c4, carry2):
            for cp in range(4):
                xrow = xnumf_v[pl.ds(c4 * 4 * L + cp * L, L)]
                for d in range(D_NUM):
                    sv = jnp.full((L,), xrow[d], jnp.float32)
                    tok = 39 * cp + d
                    for h in range(2):
                        stage_v[stage0 + 39 * c4 + tok // 4,
                                pl.ds((tok % 4) * D_TOKEN + h * L, L)] = (
                            sv * wnum[2 * d + h] + bnum[2 * d + h])
            return carry2

        lax.fori_loop(0, C // 4, num_body, 0)

        bcat = [bias_v[pl.ds((D_NUM + j) * D_TOKEN + h * L, L)]
                for j in range(N_CAT) for h in range(2)]

        def cat_body(c4, carry2):
            p0 = c4 * 4 * N_CAT
            qv = [qmod_v[pl.ds(p0 + t * L, L)] for t in range(7)]
            for cp in range(4):
                for j in range(N_CAT):
                    i = cp * N_CAT + j
                    bq = jnp.full((L,), qv[i // L][i % L], jnp.int32)
                    flo = lax.bitwise_and(bq, 1).astype(jnp.float32)
                    fhi = lax.shift_right_logical(bq, 1).astype(jnp.float32)
                    tok = 39 * cp + D_NUM + j
                    for h in range(2):
                        v0 = rows_v[p0 + i, pl.ds(h * L, L)]
                        v1 = rows_v[p0 + i, pl.ds(D_TOKEN + h * L, L)]
                        v2 = rows_v[p0 + i, pl.ds(2 * D_TOKEN + h * L, L)]
                        v3 = rows_v[p0 + i, pl.ds(3 * D_TOKEN + h * L, L)]
                        s01 = v0 + flo * (v1 - v0)
                        s23 = v2 + flo * (v3 - v2)
                        v = s01 + fhi * (s23 - s01)
                        stage_v[stage0 + 39 * c4 + tok // 4,
                                pl.ds((tok % 4) * D_TOKEN + h * L, L)] = (
                            v + bcat[2 * j + h])
            return carry2

        lax.fori_loop(0, C // 4, cat_body, 0)

    def chunk_body(g2, carry):
        base = wid * BPW + g2 * 2 * C
        sub_chunk(base, 0)
        sub_chunk(base + C, CHUNK_LINES)
        row0 = pl.multiple_of(base * N_TOK // 4, 8)
        pltpu.sync_copy(stage_v, out_hbm.at[pl.ds(row0, 2 * CHUNK_LINES)])
        return carry

    lax.fori_loop(0, G2, chunk_body, 0)


@jax.jit
def _encoder(x_numf, x_catf, weightf, table2, biasf, offs_tile):
    mesh = plsc.VectorSubcoreMesh(core_axis_name="c", subcore_axis_name="s")
    f = pl.kernel(
        _encoder_body, mesh=mesh,
        compiler_params=pltpu.CompilerParams(use_tc_tiling_on_sc=True),
        out_type=jax.ShapeDtypeStruct((BATCH * N_TOK // 4, LINE), jnp.float32),
        scratch_types=[
            pltpu.VMEM((R,), jnp.int32),
            pltpu.VMEM((C * L,), jnp.float32),
            pltpu.VMEM((R,), jnp.int32),
            pltpu.VMEM((D_NUM * D_TOKEN,), jnp.float32),
            pltpu.VMEM((N_TOK * D_TOKEN,), jnp.float32),
            pltpu.VMEM((N_DMA, LINE), jnp.int32),
            pltpu.VMEM((QPAD,), jnp.int32),
            pltpu.VMEM((N_DMA * LINE, LINE), jnp.float32),
            pltpu.VMEM((2 * CHUNK_LINES, LINE), jnp.float32),
            pltpu.SemaphoreType.DMA,
        ],
    )
    return f(x_numf, x_catf, weightf, table2, biasf, offs_tile)


def kernel(x_num, x_cat, weight, cat_table, tab_bias, category_offsets):
    x_numf = jnp.pad(x_num, ((0, 0), (0, L - D_NUM))).reshape(BATCH * L)
    x_catf = x_cat.reshape(BATCH * N_CAT)
    table2 = cat_table.reshape(TABLE_LINES, LINE)
    weightf = weight.reshape(D_NUM * D_TOKEN)
    biasf = tab_bias.reshape(N_TOK * D_TOKEN)
    offs_tile = jnp.tile(category_offsets.astype(jnp.int32), C)
    out = _encoder(x_numf, x_catf, weightf, table2, biasf, offs_tile)
    return out.reshape(BATCH, N_TOK, D_TOKEN)


# speedup vs baseline: 3.0870x; 1.4110x over previous
import jax
import jax.numpy as jnp
from jax import lax
from jax.experimental import pallas as pl
from jax.experimental.pallas import tpu as pltpu
from jax.experimental.pallas import tpu_sc as plsc

BATCH = 16384
D_NUM = 13
N_CAT = 26
D_TOKEN = 32
N_TOK = D_NUM + N_CAT
TABLE_ROWS = 2600000

_info = plsc.get_sparse_core_info()
NC, NS, L = _info.num_cores, _info.num_subcores, _info.num_lanes
NW = NC * NS
BPW = BATCH // NW

C = 32
G = BPW // C
R = C * N_CAT
DMA_ROWS = 64
N_DMA = R // DMA_ROWS


def _encoder_body(xnumf_hbm, xcatf_hbm, weightf_hbm, table_hbm, biasf_hbm,
                  offs_hbm, out_hbm,
                  xcatf_v, xnumf_v, offs_v, weight_v, bias_v, idx_v, rows_v,
                  stage_v, sem):
    wid = lax.axis_index("s") * NC + lax.axis_index("c")

    pltpu.sync_copy(offs_hbm, offs_v)
    pltpu.sync_copy(weightf_hbm, weight_v)
    pltpu.sync_copy(biasf_hbm, bias_v)

    def chunk_body(g, carry):
        base = wid * BPW + g * C

        pltpu.sync_copy(xcatf_hbm.at[pl.ds(base * N_CAT, R)], xcatf_v)
        pltpu.sync_copy(xnumf_hbm.at[pl.ds(base * L, C * L)], xnumf_v)

        for r in range(N_DMA):
            for q in range(DMA_ROWS // L):
                p = r * DMA_ROWS + q * L
                idx_v[r, pl.ds(q * L, L)] = (
                    xcatf_v[pl.ds(p, L)] + offs_v[pl.ds(p, L)])

        handles = [
            pltpu.async_copy(table_hbm.at[idx_v.at[r]],
                             rows_v.at[pl.ds(r * DMA_ROWS, DMA_ROWS)], sem)
            for r in range(N_DMA)
        ]
        for h in handles:
            h.wait()

        wnum = [weight_v[pl.ds(d * D_TOKEN + h * L, L)]
                for d in range(D_NUM) for h in range(2)]
        bnum = [bias_v[pl.ds(d * D_TOKEN + h * L, L)]
                for d in range(D_NUM) for h in range(2)]

        def num_body(c, carry2):
            xrow = xnumf_v[pl.ds(c * L, L)]
            for d in range(D_NUM):
                sv = jnp.full((L,), xrow[d], jnp.float32)
                for h in range(2):
                    stage_v[c, d, pl.ds(h * L, L)] = (
                        sv * wnum[2 * d + h] + bnum[2 * d + h])
            return carry2

        lax.fori_loop(0, C, num_body, 0)

        bcat = [bias_v[pl.ds((D_NUM + j) * D_TOKEN + h * L, L)]
                for j in range(N_CAT) for h in range(2)]

        def cat_body(c, carry2):
            src0 = c * N_CAT
            for j in range(N_CAT):
                for h in range(2):
                    stage_v[c, D_NUM + j, pl.ds(h * L, L)] = (
                        rows_v[src0 + j, pl.ds(h * L, L)] + bcat[2 * j + h])
            return carry2

        lax.fori_loop(0, C, cat_body, 0)

        pltpu.sync_copy(stage_v, out_hbm.at[pl.ds(base, C)])
        return carry

    lax.fori_loop(0, G, chunk_body, 0)


@jax.jit
def _encoder(x_numf, x_catf, weightf, table, biasf, offs_tile):
    mesh = plsc.VectorSubcoreMesh(core_axis_name="c", subcore_axis_name="s")
    f = pl.kernel(
        _encoder_body, mesh=mesh,
        compiler_params=pltpu.CompilerParams(use_tc_tiling_on_sc=False),
        out_type=jax.ShapeDtypeStruct((BATCH, N_TOK, D_TOKEN), jnp.float32),
        scratch_types=[
            pltpu.VMEM((R,), jnp.int32),
            pltpu.VMEM((C * L,), jnp.float32),
            pltpu.VMEM((R,), jnp.int32),
            pltpu.VMEM((D_NUM * D_TOKEN,), jnp.float32),
            pltpu.VMEM((N_TOK * D_TOKEN,), jnp.float32),
            pltpu.VMEM((N_DMA, DMA_ROWS), jnp.int32),
            pltpu.VMEM((R, D_TOKEN), jnp.float32),
            pltpu.VMEM((C, N_TOK, D_TOKEN), jnp.float32),
            pltpu.SemaphoreType.DMA,
        ],
    )
    return f(x_numf, x_catf, weightf, table, biasf, offs_tile)


def kernel(x_num, x_cat, weight, cat_table, tab_bias, category_offsets):
    x_numf = jnp.pad(x_num, ((0, 0), (0, L - D_NUM))).reshape(BATCH * L)
    x_catf = x_cat.reshape(BATCH * N_CAT)
    weightf = weight.reshape(D_NUM * D_TOKEN)
    biasf = tab_bias.reshape(N_TOK * D_TOKEN)
    offs_tile = jnp.tile(category_offsets.astype(jnp.int32), C)
    return _encoder(x_numf, x_catf, weightf, cat_table, biasf, offs_tile)
